# baseline (speedup 1.0000x reference)
import functools

import jax
import jax.numpy as jnp
from jax import lax
from jax.experimental import pallas as pl
from jax.experimental.pallas import tpu as pltpu
from jax.experimental.pallas import tpu_sc as plsc

TOPK = 5
BATCH_N = 128
VOCAB_N = 100000
LANES = 16
NCORES = 2
NSUB = 16
NWORKERS = NCORES * NSUB
ROWS_PER = BATCH_N // NWORKERS
VECS = VOCAB_N // LANES
LAB_PAD = 32
BIGPOS = jnp.int32(1 << 30)


def _row_topk_insert(i, carry, row_ref):
    t0, t1, t2, t3, t4 = carry
    v = row_ref[pl.ds(i * LANES, LANES)]
    n0 = jnp.maximum(t0, v)
    r = jnp.minimum(t0, v)
    n1 = jnp.maximum(t1, r)
    r = jnp.minimum(t1, r)
    n2 = jnp.maximum(t2, r)
    r = jnp.minimum(t2, r)
    n3 = jnp.maximum(t3, r)
    r = jnp.minimum(t3, r)
    n4 = jnp.maximum(t4, r)
    return n0, n1, n2, n3, n4


def _merge_t5(merge_v):
    iota = lax.iota(jnp.int32, LANES)

    def sel_iter(_, t5_prev):
        mv = jnp.full((LANES,), -jnp.inf, jnp.float32)
        mp = jnp.full((LANES,), BIGPOS, jnp.int32)
        for j in range(TOPK):
            cur = merge_v[j]
            p = iota + jnp.int32(j * LANES)
            upd = (cur > mv) | ((cur == mv) & (p < mp))
            mv = jnp.where(upd, cur, mv)
            mp = jnp.where(upd, p, mp)
        mvs = jnp.max(mv)
        mps = jnp.min(jnp.where(mv == mvs, mp, BIGPOS))
        for j in range(TOPK):
            cur = merge_v[j]
            p = iota + jnp.int32(j * LANES)
            merge_v[j] = jnp.where(p == mps, -jnp.inf, cur)
        return mvs

    return lax.fori_loop(0, TOPK, sel_iter, jnp.float32(0.0))


@functools.partial(
    pl.kernel,
    out_type=jax.ShapeDtypeStruct((NCORES, LANES), jnp.float32),
    mesh=plsc.VectorSubcoreMesh(core_axis_name="c", subcore_axis_name="s"),
    compiler_params=pltpu.CompilerParams(needs_layout_passes=False),
    scratch_types=[
        pltpu.VMEM((VOCAB_N,), jnp.float32),
        pltpu.VMEM((LAB_PAD,), jnp.int32),
        pltpu.VMEM((TOPK, LANES), jnp.float32),
        pltpu.VMEM((LANES,), jnp.float32),
        pltpu.VMEM((NSUB, LANES), jnp.float32),
        pltpu.VMEM((LANES,), jnp.float32),
        pltpu.VMEM_SHARED((NSUB, LANES), jnp.float32),
    ],
)
def _sc_topk_acc(probas_hbm, labels_hbm, out_hbm,
                 row_v, lab_v, merge_v, cnt_v, sums_v, out_v, shared):
    core = lax.axis_index("c")
    sid = lax.axis_index("s")
    wid = sid * NCORES + core

    def row_body(i, count):
        r = wid * ROWS_PER + i
        pltpu.sync_copy(probas_hbm.at[r], row_v)
        pltpu.sync_copy(labels_hbm.at[r], lab_v)

        neg = jnp.full((LANES,), -jnp.inf, jnp.float32)
        tops = lax.fori_loop(
            0, VECS,
            lambda j, c: _row_topk_insert(j, c, row_v),
            (neg, neg, neg, neg, neg),
        )
        for j in range(TOPK):
            merge_v[j] = tops[j]
        t5 = _merge_t5(merge_v)

        g0 = plsc.load_gather(row_v, [lab_v[pl.ds(0, LANES)]])
        g1 = plsc.load_gather(row_v, [lab_v[pl.ds(LANES, LANES)]])
        lmax = jnp.max(jnp.maximum(g0, g1))
        return count + jnp.where(lmax >= t5, jnp.float32(1.0), jnp.float32(0.0))

    count = lax.fori_loop(0, ROWS_PER, row_body, jnp.float32(0.0))

    cnt_v[...] = jnp.broadcast_to(count, (LANES,))
    pltpu.sync_copy(cnt_v, shared.at[sid])
    plsc.subcore_barrier()

    @pl.when(sid == 0)
    def _():
        pltpu.sync_copy(shared, sums_v)
        tot = sums_v[0]
        for j in range(1, NSUB):
            tot = tot + sums_v[j]
        out_v[...] = tot
        pltpu.sync_copy(out_v, out_hbm.at[core])


def kernel(probas, labels):
    lab32 = jnp.concatenate(
        [labels, jnp.broadcast_to(labels[:, :1], (BATCH_N, LAB_PAD - labels.shape[1]))],
        axis=1,
    )
    out = _sc_topk_acc(probas, lab32)
    return (out[0, 0] + out[1, 0]) * jnp.float32(1.0 / BATCH_N)
